# baseline (speedup 1.0000x reference)
import functools

import jax
import jax.numpy as jnp
from jax import lax
from jax.experimental import pallas as pl
from jax.experimental.pallas import tpu as pltpu
from jax.experimental.pallas import tpu_sc as plsc

N = 100000
D = 31
DP = 32
H = 16
E = 1600000
LANES = 512
LANES8 = 128
EPAD = 1638400
ROWS = EPAD // LANES
KB = 2
RPT = ROWS // 16
NBLK2 = RPT // 2
DRPT = ROWS // 32
DRND = DRPT // 2
NP = 100352
NT = NP // 16
L = 5
G = 10
MAXL = 120 * 31
RB = 1000
NRB = N // RB

_mesh = plsc.VectorSubcoreMesh(core_axis_name="c", subcore_axis_name="s")


def _make_hop(scale):
    out_type = [jax.ShapeDtypeStruct((2, NP, H), jnp.float32)]
    if scale:
        out_type.append(jax.ShapeDtypeStruct((2, NP, H), jnp.float32))

    def hop_body(*args):
        if scale:
            (t0, t1, srcp, dstp, z16, n2b, agg, tout,
             sp, srcbb, dstbb, rows0, rows1, g0s, g1s, s0s, s1s) = args
        else:
            (t0, t1, srcp, dstp, z16, agg,
             sp, srcbb, dstbb, rows0, rows1, g0s, g1s, s0s, s1s) = args
        c = lax.axis_index("c")
        s = lax.axis_index("s")

        pltpu.sync_copy(z16.at[pl.ds(0, NT), :], sp.at[pl.ds(s * NT, NT), :])
        plsc.subcore_barrier()

        def run(t):
            base = s * RPT
            dummy = t.at[pl.ds(0, LANES), :]

            def wait(sem, buf):
                pltpu.make_async_copy(dummy, buf, sem).wait()

            def sup(S, carry):
                row0 = base + S * 10
                pltpu.sync_copy(srcp.at[pl.ds(row0, 10), :], srcbb)
                pltpu.sync_copy(dstp.at[pl.ds(row0, 10), :], dstbb)

                @pl.when(S > 0)
                def _():
                    wait(s0s, rows0)
                pltpu.async_copy(t.at[srcbb.at[0]], rows0, g0s)

                for j in range(10):
                    cur = rows0 if j % 2 == 0 else rows1
                    oth = rows1 if j % 2 == 0 else rows0
                    gcur = g0s if j % 2 == 0 else g1s
                    goth = g1s if j % 2 == 0 else g0s
                    soth = s1s if j % 2 == 0 else s0s
                    scur = s0s if j % 2 == 0 else s1s
                    if j < 9:
                        if j == 0:
                            @pl.when(S > 0)
                            def _():
                                wait(soth, oth)
                        else:
                            wait(soth, oth)
                        pltpu.async_copy(t.at[srcbb.at[j + 1]], oth, goth)
                    wait(gcur, cur)
                    pltpu.async_copy(cur, sp.at[dstbb.at[j]], scur, add=True)
                return carry

            lax.fori_loop(0, RPT // 10, sup, 0)
            wait(s0s, rows0)
            wait(s1s, rows1)

        @pl.when(c == 0)
        def _():
            run(t0)

        @pl.when(c == 1)
        def _():
            run(t1)

        plsc.subcore_barrier()
        pltpu.sync_copy(sp.at[pl.ds(s * NT, NT), :], agg.at[c, pl.ds(s * NT, NT), :])
        if scale:
            CH = 448
            for q in range(NT // CH):
                r0 = s * NT + q * CH
                pltpu.sync_copy(sp.at[pl.ds(r0, CH), :], rows0.at[pl.ds(0, CH), :])
                pltpu.sync_copy(n2b.at[pl.ds(r0, CH), :], rows1.at[pl.ds(0, CH), :])

                def mul8(i, carry):
                    for u in range(8):
                        r = i * 8 + u
                        rows0[r, :] = rows0[r, :] * rows1[r, :]
                    return carry

                lax.fori_loop(0, CH // 8, mul8, 0)
                pltpu.sync_copy(rows0.at[pl.ds(0, CH), :], tout.at[c, pl.ds(r0, CH), :])

    return functools.partial(
        pl.kernel,
        out_type=out_type if scale else out_type[0],
        mesh=_mesh,
        compiler_params=pltpu.CompilerParams(use_tc_tiling_on_sc=False),
        scratch_types=[
            pltpu.VMEM_SHARED((NP, H), jnp.float32),
            pltpu.VMEM((10, LANES), jnp.int32),
            pltpu.VMEM((10, LANES), jnp.int32),
            pltpu.VMEM((LANES, H), jnp.float32),
            pltpu.VMEM((LANES, H), jnp.float32),
            pltpu.SemaphoreType.DMA,
            pltpu.SemaphoreType.DMA,
            pltpu.SemaphoreType.DMA,
            pltpu.SemaphoreType.DMA,
        ],
    )(hop_body)


_hop_scaled = _make_hop(True)
_hop_kernel = _make_hop(False)


@functools.partial(
    pl.kernel,
    out_type=[
        jax.ShapeDtypeStruct((NP,), jnp.float32),
        jax.ShapeDtypeStruct((NP,), jnp.float32),
    ],
    mesh=_mesh,
    compiler_params=pltpu.CompilerParams(use_tc_tiling_on_sc=False),
    scratch_types=[
        pltpu.VMEM_SHARED((NP,), jnp.float32),
        pltpu.VMEM((1, LANES), jnp.int32),
        pltpu.VMEM((1, LANES), jnp.int32),
        pltpu.VMEM((LANES,), jnp.float32),
        pltpu.SemaphoreType.DMA,
        pltpu.SemaphoreType.DMA,
    ],
)
def _deg_kernel(dstp, z1, deg0, deg1, dsp, dstb0, dstb1, ones, s0, s1):
    c = lax.axis_index("c")
    s = lax.axis_index("s")
    for i in range(LANES // 16):
        ones[pl.ds(i * 16, 16)] = jnp.ones((16,), jnp.float32)

    pltpu.sync_copy(z1.at[pl.ds(0, NT)], dsp.at[pl.ds(s * NT, NT)])
    plsc.subcore_barrier()

    base = (c * 16 + s) * DRPT
    zdum = z1.at[pl.ds(0, LANES)]

    def wait(sem):
        pltpu.make_async_copy(zdum, ones, sem).wait()

    pltpu.sync_copy(dstp.at[pl.ds(base, 1), :], dstb0)

    def blk(k, carry):
        @pl.when(k > 0)
        def _():
            wait(s1)
        pltpu.sync_copy(dstp.at[pl.ds(base + 2 * k + 1, 1), :], dstb1)
        pltpu.async_copy(ones, dsp.at[dstb0.at[0]], s0, add=True)

        @pl.when(k < DRND - 1)
        def _():
            wait(s0)
            pltpu.sync_copy(dstp.at[pl.ds(base + 2 * k + 2, 1), :], dstb0)
        pltpu.async_copy(ones, dsp.at[dstb1.at[0]], s1, add=True)
        return carry

    lax.fori_loop(0, DRND, blk, 0)
    wait(s0)
    wait(s1)
    plsc.subcore_barrier()

    def wb(deg):
        pltpu.sync_copy(dsp.at[pl.ds(s * NT, NT)], deg.at[pl.ds(s * NT, NT)])

    @pl.when(c == 0)
    def _():
        wb(deg0)

    @pl.when(c == 1)
    def _():
        wb(deg1)


PR = NP // 8
PBR = 128
NB = PR // PBR
W32 = 256


def _prep_body(degb_ref, degb32_ref, x0_ref, x1_ref,
               normp_ref, norm2p_ref, norm32_ref, h0_ref, h1_ref):
    nrm = lax.rsqrt(jnp.maximum(degb_ref[...], 1.0))
    normp_ref[...] = nrm
    norm2p_ref[...] = nrm * nrm
    norm32_ref[...] = lax.rsqrt(jnp.maximum(degb32_ref[...], 1.0))
    h0_ref[...] = x0_ref[...] * nrm
    h1_ref[...] = x1_ref[...] * nrm


def _pk(w=LANES8):
    return pl.BlockSpec((PBR, w), lambda i: (i, 0))


def _prep_call(degb, degb32, x0p, x1p):
    return pl.pallas_call(
        _prep_body,
        grid=(NB,),
        in_specs=[_pk(), _pk(W32), _pk(), _pk()],
        out_specs=[_pk(), _pk(), _pk(W32), _pk(), _pk()],
        out_shape=[
            jax.ShapeDtypeStruct((PR, LANES8), jnp.float32),
            jax.ShapeDtypeStruct((PR, LANES8), jnp.float32),
            jax.ShapeDtypeStruct((PR, W32), jnp.float32),
            jax.ShapeDtypeStruct((PR, LANES8), jnp.float32),
            jax.ShapeDtypeStruct((PR, LANES8), jnp.float32),
        ],
    )(degb, degb32, x0p, x1p)


def _mid_body(agg_ref, norm2p_ref, t0_ref, t1_ref):
    n2 = norm2p_ref[...]
    t0_ref[...] = agg_ref[0] * n2
    t1_ref[...] = agg_ref[1] * n2


def _mid_call(aggP, norm2p):
    return pl.pallas_call(
        _mid_body,
        grid=(NB,),
        in_specs=[pl.BlockSpec((2, PBR, LANES8), lambda i: (0, i, 0)), _pk()],
        out_specs=[_pk(), _pk()],
        out_shape=[jax.ShapeDtypeStruct((PR, LANES8), jnp.float32)] * 2,
    )(aggP, norm2p)


def _layer_body(hP_ref, aA_ref, aB_ref, np16_ref, np32_ref,
                kw0_ref, kw1t_ref, kw1b_ref, kw2t_ref, kw2b_ref,
                bP_ref, sel0_ref, sel1_ref,
                hn_ref, o0_ref, o1_ref):
    n16 = np16_ref[...]
    dot = lambda a, b: jnp.dot(a, b, preferred_element_type=jnp.float32)
    acc = dot(hP_ref[...], kw0_ref[...])
    acc = acc + dot(aA_ref[0] * n16, kw1t_ref[...])
    acc = acc + dot(aA_ref[1] * n16, kw1b_ref[...])
    acc = acc + dot(aB_ref[0] * n16, kw2t_ref[...])
    acc = acc + dot(aB_ref[1] * n16, kw2b_ref[...])
    acc = jnp.maximum(acc + bP_ref[0:1, :], 0.0)
    hn_ref[...] = acc
    sc = acc * np32_ref[...]
    o0_ref[...] = dot(sc, sel0_ref[...])
    o1_ref[...] = dot(sc, sel1_ref[...])


def _layer_call(hP, aggAP, aggBP, normp, norm32p, kws, bP, sel0, sel1):
    pk2 = pl.BlockSpec((2, PBR, LANES8), lambda i: (0, i, 0))
    fw = lambda a, b: pl.BlockSpec((a, b), lambda i: (0, 0))
    return pl.pallas_call(
        _layer_body,
        grid=(NB,),
        in_specs=[
            _pk(W32), pk2, pk2, _pk(), _pk(W32),
            fw(W32, W32), fw(LANES8, W32), fw(LANES8, W32),
            fw(LANES8, W32), fw(LANES8, W32),
            fw(8, W32), fw(W32, LANES8), fw(W32, LANES8),
        ],
        out_specs=[_pk(W32), _pk(), _pk()],
        out_shape=[
            jax.ShapeDtypeStruct((PR, W32), jnp.float32),
            jax.ShapeDtypeStruct((PR, LANES8), jnp.float32),
            jax.ShapeDtypeStruct((PR, LANES8), jnp.float32),
        ],
    )(hP, aggAP, aggBP, normp, norm32p, *kws, bP, sel0, sel1)


def _gate_body(hP_ref, gid8_ref, kgw_ref, gmax_ref, acc_ref):
    i = pl.program_id(0)

    @pl.when(i == 0)
    def _():
        acc_ref[...] = jnp.full((16, LANES8), -3e38, jnp.float32)

    g8 = jnp.dot(hP_ref[...], kgw_ref[...], preferred_element_type=jnp.float32)
    gid = gid8_ref[...]
    mx = [jnp.max(jnp.where(gid == gg, g8, -3e38)) for gg in range(G)]
    vec = jnp.concatenate([jnp.stack(mx), jnp.full((16 - G,), -3e38, jnp.float32)])
    acc_ref[...] = jnp.maximum(acc_ref[...], vec[:, None] + jnp.zeros((16, LANES8), jnp.float32))

    @pl.when(i == pl.num_programs(0) - 1)
    def _():
        gmax_ref[...] = acc_ref[...]


def _gate_call(hP, gid8, kgw):
    return pl.pallas_call(
        _gate_body,
        grid=(NB,),
        in_specs=[
            _pk(W32),
            pl.BlockSpec((PBR, 8), lambda i: (i, 0)),
            pl.BlockSpec((W32, 8), lambda i: (0, 0)),
        ],
        out_specs=[pl.BlockSpec((16, LANES8), lambda i: (0, 0))],
        out_shape=[jax.ShapeDtypeStruct((16, LANES8), jnp.float32)],
        scratch_shapes=[pltpu.VMEM((16, LANES8), jnp.float32)],
    )(hP, gid8, kgw)


def _pool_body(hP_ref, gid8_ref, kgw_ref, bc32_ref, fold_ref, gmax_ref,
               rep_ref, s_ref, racc, sacc):
    i = pl.program_id(0)

    @pl.when(i == 0)
    def _():
        racc[...] = jnp.zeros((16, DP), jnp.float32)
        sacc[...] = jnp.zeros((16, LANES8), jnp.float32)

    dot = lambda a, b: jnp.dot(a, b, preferred_element_type=jnp.float32)
    hP = hP_ref[...]
    g8 = dot(hP, kgw_ref[...])
    gid = gid8_ref[...]
    ss = []
    reps = []
    for gg in range(G):
        e8 = jnp.where(gid == gg, jnp.exp(g8 - gmax_ref[gg, 0]), 0.0)
        ss.append(jnp.sum(e8))
        eb32 = dot(e8, bc32_ref[...])
        col = jnp.sum(eb32 * hP, axis=0)
        reps.append(dot(col[None, :], fold_ref[...])[0])
    svec = jnp.concatenate([jnp.stack(ss), jnp.zeros((16 - G,), jnp.float32)])
    rmat = jnp.pad(jnp.stack(reps), ((0, 16 - G), (0, 0)))
    sacc[...] = sacc[...] + svec[:, None] + jnp.zeros((16, LANES8), jnp.float32)
    racc[...] = racc[...] + rmat

    @pl.when(i == pl.num_programs(0) - 1)
    def _():
        rep_ref[...] = racc[...]
        s_ref[...] = sacc[...]


def _pool_call(hP, gid8, kgw, bc32, fold, gmax):
    return pl.pallas_call(
        _pool_body,
        grid=(NB,),
        in_specs=[
            _pk(W32),
            pl.BlockSpec((PBR, 8), lambda i: (i, 0)),
            pl.BlockSpec((W32, 8), lambda i: (0, 0)),
            pl.BlockSpec((8, W32), lambda i: (0, 0)),
            pl.BlockSpec((W32, DP), lambda i: (0, 0)),
            pl.BlockSpec((16, LANES8), lambda i: (0, 0)),
        ],
        out_specs=[
            pl.BlockSpec((16, DP), lambda i: (0, 0)),
            pl.BlockSpec((16, LANES8), lambda i: (0, 0)),
        ],
        out_shape=[
            jax.ShapeDtypeStruct((16, DP), jnp.float32),
            jax.ShapeDtypeStruct((16, LANES8), jnp.float32),
        ],
        scratch_shapes=[
            pltpu.VMEM((16, DP), jnp.float32),
            pltpu.VMEM((16, LANES8), jnp.float32),
        ],
    )(hP, gid8, kgw, bc32, fold, gmax)


def kernel(x, edge_index, graph_ids, num, W, b, gate_W, gate_b):
    src = edge_index[0]
    dst = edge_index[1]
    srcp = jnp.concatenate([src, jnp.zeros((EPAD - E,), jnp.int32)]).reshape(ROWS, LANES)
    dstp = jnp.concatenate([dst, jnp.full((EPAD - E,), N, jnp.int32)]).reshape(ROWS, LANES)
    z16 = jnp.zeros((NT, H), jnp.float32)
    z1 = jnp.zeros((NT,), jnp.float32)

    xp = jnp.pad(x, ((0, NP - N), (0, DP - D)))
    x0p = xp[:, :H].reshape(PR, LANES8)
    x1p = xp[:, H:].reshape(PR, LANES8)
    hP = xp.reshape(PR, W32)
    gid_pad = jnp.pad(graph_ids, (0, NP - N), constant_values=G)
    gid8 = gid_pad.reshape(PR, 8)

    deg0, deg1 = _deg_kernel(dstp, z1)
    degsum = deg0 + deg1
    degb = jnp.repeat(degsum, H).reshape(PR, LANES8)
    degb32 = jnp.repeat(degsum, DP).reshape(PR, W32)
    normp, norm2p, norm32p, h0p, h1p = _prep_call(degb, degb32, x0p, x1p)

    eye8 = jnp.eye(8, dtype=jnp.float32)
    sel0 = jnp.kron(eye8, jnp.concatenate(
        [jnp.eye(H, dtype=jnp.float32), jnp.zeros((H, H), jnp.float32)], axis=0))
    sel1 = jnp.kron(eye8, jnp.concatenate(
        [jnp.zeros((H, H), jnp.float32), jnp.eye(H, dtype=jnp.float32)], axis=0))

    for l in range(L):
        aggA, tA = _hop_scaled(h0p.reshape(NP, H), h1p.reshape(NP, H), srcp, dstp,
                               z16, norm2p.reshape(NP, H))
        aggB = _hop_kernel(tA[0], tA[1], srcp, dstp, z16)
        w0 = jnp.pad(W[l, 0:D, :], ((0, DP - D), (0, DP - D)))
        w1 = jnp.pad(W[l, D:2 * D, :], ((0, DP - D), (0, DP - D)))
        w2 = jnp.pad(W[l, 2 * D:3 * D, :], ((0, DP - D), (0, DP - D)))
        kws = [
            jnp.kron(eye8, w0),
            jnp.kron(eye8, w1[:H, :]), jnp.kron(eye8, w1[H:, :]),
            jnp.kron(eye8, w2[:H, :]), jnp.kron(eye8, w2[H:, :]),
        ]
        bP = jnp.tile(jnp.pad(b[l], (0, DP - D)), (8, 8)).reshape(8, 8 * DP)[:, :W32]
        hP, h0p, h1p = _layer_call(hP, aggA.reshape(2, PR, LANES8),
                                   aggB.reshape(2, PR, LANES8), normp, norm32p,
                                   kws, bP, sel0, sel1)

    gwp = jnp.pad(gate_W[:, 0], (0, DP - D))
    kgw = jnp.kron(eye8, gwp[:, None])
    bc32 = jnp.kron(eye8, jnp.ones((1, DP), jnp.float32))
    fold = jnp.tile(jnp.eye(DP, dtype=jnp.float32), (8, 1))
    (gmax,) = _gate_call(hP, gid8, kgw)
    rep, ssum = _pool_call(hP, gid8, kgw, bc32, fold, gmax)
    rep10 = rep[:G, :D] / ssum[:G, 0:1]

    rep_flat = rep10.reshape(-1)
    offsets = (jnp.cumsum(num) - num) * D
    flat_len = (num.shape[0] // G) * D

    def _take(o):
        row = lax.dynamic_slice(rep_flat, (o,), (flat_len,))
        return jnp.pad(row, (0, MAXL - flat_len))

    return jax.vmap(_take)(offsets)


# speedup vs baseline: 1.1840x; 1.1840x over previous
import functools

import jax
import jax.numpy as jnp
from jax import lax
from jax.experimental import pallas as pl
from jax.experimental.pallas import tpu as pltpu
from jax.experimental.pallas import tpu_sc as plsc

N = 100000
D = 31
DP = 32
H = 16
E = 1600000
LANES = 512
LANES8 = 128
EPAD = 1638400
ROWS = EPAD // LANES
KB = 2
RPT = ROWS // 16
NBLK2 = RPT // 2
DRPT = ROWS // 32
DRND = DRPT // 2
NP = 100352
NT = NP // 16
L = 5
G = 10
MAXL = 120 * 31
RB = 1000
NRB = N // RB

_mesh = plsc.VectorSubcoreMesh(core_axis_name="c", subcore_axis_name="s")


@functools.partial(
    pl.kernel,
    out_type=jax.ShapeDtypeStruct((2, NP, H), jnp.float32),
    mesh=_mesh,
    compiler_params=pltpu.CompilerParams(use_tc_tiling_on_sc=False),
    scratch_types=[
        pltpu.VMEM_SHARED((NP, H), jnp.float32),
        pltpu.VMEM((10, LANES), jnp.int32),
        pltpu.VMEM((10, LANES), jnp.int32),
        pltpu.VMEM((LANES, H), jnp.float32),
        pltpu.VMEM((LANES, H), jnp.float32),
        pltpu.SemaphoreType.DMA,
        pltpu.SemaphoreType.DMA,
        pltpu.SemaphoreType.DMA,
        pltpu.SemaphoreType.DMA,
    ],
)
def _hop_kernel(t0, t1, srcp, dstp, z16, agg,
                sp, srcbb, dstbb, rows0, rows1,
                g0s, g1s, s0s, s1s):
    c = lax.axis_index("c")
    s = lax.axis_index("s")

    pltpu.sync_copy(z16.at[pl.ds(0, NT), :], sp.at[pl.ds(s * NT, NT), :])
    plsc.subcore_barrier()

    def run(t):
        base = s * RPT
        dummy = t.at[pl.ds(0, LANES), :]

        def wait(sem, buf):
            pltpu.make_async_copy(dummy, buf, sem).wait()

        def sup(S, carry):
            row0 = base + S * 10
            pltpu.sync_copy(srcp.at[pl.ds(row0, 10), :], srcbb)
            pltpu.sync_copy(dstp.at[pl.ds(row0, 10), :], dstbb)

            @pl.when(S > 0)
            def _():
                wait(s0s, rows0)
            pltpu.async_copy(t.at[srcbb.at[0]], rows0, g0s)

            for j in range(10):
                cur = rows0 if j % 2 == 0 else rows1
                oth = rows1 if j % 2 == 0 else rows0
                gcur = g0s if j % 2 == 0 else g1s
                goth = g1s if j % 2 == 0 else g0s
                soth = s1s if j % 2 == 0 else s0s
                scur = s0s if j % 2 == 0 else s1s
                if j < 9:
                    if j == 0:
                        @pl.when(S > 0)
                        def _():
                            wait(soth, oth)
                    else:
                        wait(soth, oth)
                    pltpu.async_copy(t.at[srcbb.at[j + 1]], oth, goth)
                wait(gcur, cur)
                pltpu.async_copy(cur, sp.at[dstbb.at[j]], scur, add=True)
            return carry

        lax.fori_loop(0, RPT // 10, sup, 0)
        wait(s0s, rows0)
        wait(s1s, rows1)

    @pl.when(c == 0)
    def _():
        run(t0)

    @pl.when(c == 1)
    def _():
        run(t1)

    plsc.subcore_barrier()
    pltpu.sync_copy(sp.at[pl.ds(s * NT, NT), :], agg.at[c, pl.ds(s * NT, NT), :])


@functools.partial(
    pl.kernel,
    out_type=[
        jax.ShapeDtypeStruct((NP,), jnp.float32),
        jax.ShapeDtypeStruct((NP,), jnp.float32),
    ],
    mesh=_mesh,
    compiler_params=pltpu.CompilerParams(use_tc_tiling_on_sc=False),
    scratch_types=[
        pltpu.VMEM_SHARED((NP,), jnp.float32),
        pltpu.VMEM((1, LANES), jnp.int32),
        pltpu.VMEM((1, LANES), jnp.int32),
        pltpu.VMEM((LANES,), jnp.float32),
        pltpu.SemaphoreType.DMA,
        pltpu.SemaphoreType.DMA,
    ],
)
def _deg_kernel(dstp, z1, deg0, deg1, dsp, dstb0, dstb1, ones, s0, s1):
    c = lax.axis_index("c")
    s = lax.axis_index("s")
    for i in range(LANES // 16):
        ones[pl.ds(i * 16, 16)] = jnp.ones((16,), jnp.float32)

    pltpu.sync_copy(z1.at[pl.ds(0, NT)], dsp.at[pl.ds(s * NT, NT)])
    plsc.subcore_barrier()

    base = (c * 16 + s) * DRPT
    zdum = z1.at[pl.ds(0, LANES)]

    def wait(sem):
        pltpu.make_async_copy(zdum, ones, sem).wait()

    pltpu.sync_copy(dstp.at[pl.ds(base, 1), :], dstb0)

    def blk(k, carry):
        @pl.when(k > 0)
        def _():
            wait(s1)
        pltpu.sync_copy(dstp.at[pl.ds(base + 2 * k + 1, 1), :], dstb1)
        pltpu.async_copy(ones, dsp.at[dstb0.at[0]], s0, add=True)

        @pl.when(k < DRND - 1)
        def _():
            wait(s0)
            pltpu.sync_copy(dstp.at[pl.ds(base + 2 * k + 2, 1), :], dstb0)
        pltpu.async_copy(ones, dsp.at[dstb1.at[0]], s1, add=True)
        return carry

    lax.fori_loop(0, DRND, blk, 0)
    wait(s0)
    wait(s1)
    plsc.subcore_barrier()

    def wb(deg):
        pltpu.sync_copy(dsp.at[pl.ds(s * NT, NT)], deg.at[pl.ds(s * NT, NT)])

    @pl.when(c == 0)
    def _():
        wb(deg0)

    @pl.when(c == 1)
    def _():
        wb(deg1)


PR = NP // 8
PBR = 128
NB = PR // PBR
W32 = 256


def _prep_body(degb_ref, degb32_ref, x0_ref, x1_ref,
               normp_ref, norm2p_ref, norm32_ref, h0_ref, h1_ref):
    nrm = lax.rsqrt(jnp.maximum(degb_ref[...], 1.0))
    normp_ref[...] = nrm
    norm2p_ref[...] = nrm * nrm
    norm32_ref[...] = lax.rsqrt(jnp.maximum(degb32_ref[...], 1.0))
    h0_ref[...] = x0_ref[...] * nrm
    h1_ref[...] = x1_ref[...] * nrm


def _pk(w=LANES8):
    return pl.BlockSpec((PBR, w), lambda i: (i, 0))


def _prep_call(degb, degb32, x0p, x1p):
    return pl.pallas_call(
        _prep_body,
        grid=(NB,),
        in_specs=[_pk(), _pk(W32), _pk(), _pk()],
        out_specs=[_pk(), _pk(), _pk(W32), _pk(), _pk()],
        out_shape=[
            jax.ShapeDtypeStruct((PR, LANES8), jnp.float32),
            jax.ShapeDtypeStruct((PR, LANES8), jnp.float32),
            jax.ShapeDtypeStruct((PR, W32), jnp.float32),
            jax.ShapeDtypeStruct((PR, LANES8), jnp.float32),
            jax.ShapeDtypeStruct((PR, LANES8), jnp.float32),
        ],
    )(degb, degb32, x0p, x1p)


def _mid_body(agg_ref, norm2p_ref, t0_ref, t1_ref):
    n2 = norm2p_ref[...]
    t0_ref[...] = agg_ref[0] * n2
    t1_ref[...] = agg_ref[1] * n2


def _mid_call(aggP, norm2p):
    return pl.pallas_call(
        _mid_body,
        grid=(NB,),
        in_specs=[pl.BlockSpec((2, PBR, LANES8), lambda i: (0, i, 0)), _pk()],
        out_specs=[_pk(), _pk()],
        out_shape=[jax.ShapeDtypeStruct((PR, LANES8), jnp.float32)] * 2,
    )(aggP, norm2p)


def _layer_body(hP_ref, aA_ref, aB_ref, np16_ref, np32_ref,
                kw0_ref, kw1t_ref, kw1b_ref, kw2t_ref, kw2b_ref,
                bP_ref, sel0_ref, sel1_ref,
                hn_ref, o0_ref, o1_ref):
    n16 = np16_ref[...]
    dot = lambda a, b: jnp.dot(a, b, preferred_element_type=jnp.float32)
    acc = dot(hP_ref[...], kw0_ref[...])
    acc = acc + dot(aA_ref[0] * n16, kw1t_ref[...])
    acc = acc + dot(aA_ref[1] * n16, kw1b_ref[...])
    acc = acc + dot(aB_ref[0] * n16, kw2t_ref[...])
    acc = acc + dot(aB_ref[1] * n16, kw2b_ref[...])
    acc = jnp.maximum(acc + bP_ref[0:1, :], 0.0)
    hn_ref[...] = acc
    sc = acc * np32_ref[...]
    o0_ref[...] = dot(sc, sel0_ref[...])
    o1_ref[...] = dot(sc, sel1_ref[...])


def _layer_call(hP, aggAP, aggBP, normp, norm32p, kws, bP, sel0, sel1):
    pk2 = pl.BlockSpec((2, PBR, LANES8), lambda i: (0, i, 0))
    fw = lambda a, b: pl.BlockSpec((a, b), lambda i: (0, 0))
    return pl.pallas_call(
        _layer_body,
        grid=(NB,),
        in_specs=[
            _pk(W32), pk2, pk2, _pk(), _pk(W32),
            fw(W32, W32), fw(LANES8, W32), fw(LANES8, W32),
            fw(LANES8, W32), fw(LANES8, W32),
            fw(8, W32), fw(W32, LANES8), fw(W32, LANES8),
        ],
        out_specs=[_pk(W32), _pk(), _pk()],
        out_shape=[
            jax.ShapeDtypeStruct((PR, W32), jnp.float32),
            jax.ShapeDtypeStruct((PR, LANES8), jnp.float32),
            jax.ShapeDtypeStruct((PR, LANES8), jnp.float32),
        ],
    )(hP, aggAP, aggBP, normp, norm32p, *kws, bP, sel0, sel1)


def _gate_body(hP_ref, gid8_ref, kgw_ref, gmax_ref, acc_ref):
    i = pl.program_id(0)

    @pl.when(i == 0)
    def _():
        acc_ref[...] = jnp.full((16, LANES8), -3e38, jnp.float32)

    g8 = jnp.dot(hP_ref[...], kgw_ref[...], preferred_element_type=jnp.float32)
    gid = gid8_ref[...]
    mx = [jnp.max(jnp.where(gid == gg, g8, -3e38)) for gg in range(G)]
    vec = jnp.concatenate([jnp.stack(mx), jnp.full((16 - G,), -3e38, jnp.float32)])
    acc_ref[...] = jnp.maximum(acc_ref[...], vec[:, None] + jnp.zeros((16, LANES8), jnp.float32))

    @pl.when(i == pl.num_programs(0) - 1)
    def _():
        gmax_ref[...] = acc_ref[...]


def _gate_call(hP, gid8, kgw):
    return pl.pallas_call(
        _gate_body,
        grid=(NB,),
        in_specs=[
            _pk(W32),
            pl.BlockSpec((PBR, 8), lambda i: (i, 0)),
            pl.BlockSpec((W32, 8), lambda i: (0, 0)),
        ],
        out_specs=[pl.BlockSpec((16, LANES8), lambda i: (0, 0))],
        out_shape=[jax.ShapeDtypeStruct((16, LANES8), jnp.float32)],
        scratch_shapes=[pltpu.VMEM((16, LANES8), jnp.float32)],
    )(hP, gid8, kgw)


def _pool_body(hP_ref, gid8_ref, kgw_ref, bc32_ref, fold_ref, gmax_ref,
               rep_ref, s_ref, racc, sacc):
    i = pl.program_id(0)

    @pl.when(i == 0)
    def _():
        racc[...] = jnp.zeros((16, DP), jnp.float32)
        sacc[...] = jnp.zeros((16, LANES8), jnp.float32)

    dot = lambda a, b: jnp.dot(a, b, preferred_element_type=jnp.float32)
    hP = hP_ref[...]
    g8 = dot(hP, kgw_ref[...])
    gid = gid8_ref[...]
    ss = []
    reps = []
    for gg in range(G):
        e8 = jnp.where(gid == gg, jnp.exp(g8 - gmax_ref[gg, 0]), 0.0)
        ss.append(jnp.sum(e8))
        eb32 = dot(e8, bc32_ref[...])
        col = jnp.sum(eb32 * hP, axis=0)
        reps.append(dot(col[None, :], fold_ref[...])[0])
    svec = jnp.concatenate([jnp.stack(ss), jnp.zeros((16 - G,), jnp.float32)])
    rmat = jnp.pad(jnp.stack(reps), ((0, 16 - G), (0, 0)))
    sacc[...] = sacc[...] + svec[:, None] + jnp.zeros((16, LANES8), jnp.float32)
    racc[...] = racc[...] + rmat

    @pl.when(i == pl.num_programs(0) - 1)
    def _():
        rep_ref[...] = racc[...]
        s_ref[...] = sacc[...]


def _pool_call(hP, gid8, kgw, bc32, fold, gmax):
    return pl.pallas_call(
        _pool_body,
        grid=(NB,),
        in_specs=[
            _pk(W32),
            pl.BlockSpec((PBR, 8), lambda i: (i, 0)),
            pl.BlockSpec((W32, 8), lambda i: (0, 0)),
            pl.BlockSpec((8, W32), lambda i: (0, 0)),
            pl.BlockSpec((W32, DP), lambda i: (0, 0)),
            pl.BlockSpec((16, LANES8), lambda i: (0, 0)),
        ],
        out_specs=[
            pl.BlockSpec((16, DP), lambda i: (0, 0)),
            pl.BlockSpec((16, LANES8), lambda i: (0, 0)),
        ],
        out_shape=[
            jax.ShapeDtypeStruct((16, DP), jnp.float32),
            jax.ShapeDtypeStruct((16, LANES8), jnp.float32),
        ],
        scratch_shapes=[
            pltpu.VMEM((16, DP), jnp.float32),
            pltpu.VMEM((16, LANES8), jnp.float32),
        ],
    )(hP, gid8, kgw, bc32, fold, gmax)


def kernel(x, edge_index, graph_ids, num, W, b, gate_W, gate_b):
    src = edge_index[0]
    dst = edge_index[1]
    srcp = jnp.concatenate([src, jnp.zeros((EPAD - E,), jnp.int32)]).reshape(ROWS, LANES)
    dstp = jnp.concatenate([dst, jnp.full((EPAD - E,), N, jnp.int32)]).reshape(ROWS, LANES)
    z16 = jnp.zeros((NT, H), jnp.float32)
    z1 = jnp.zeros((NT,), jnp.float32)

    xp = jnp.pad(x, ((0, NP - N), (0, DP - D)))
    x0p = xp[:, :H].reshape(PR, LANES8)
    x1p = xp[:, H:].reshape(PR, LANES8)
    hP = xp.reshape(PR, W32)
    gid_pad = jnp.pad(graph_ids, (0, NP - N), constant_values=G)
    gid8 = gid_pad.reshape(PR, 8)

    deg0, deg1 = _deg_kernel(dstp, z1)
    degsum = deg0 + deg1
    degb = jnp.repeat(degsum, H).reshape(PR, LANES8)
    degb32 = jnp.repeat(degsum, DP).reshape(PR, W32)
    normp, norm2p, norm32p, h0p, h1p = _prep_call(degb, degb32, x0p, x1p)

    eye8 = jnp.eye(8, dtype=jnp.float32)
    sel0 = jnp.kron(eye8, jnp.concatenate(
        [jnp.eye(H, dtype=jnp.float32), jnp.zeros((H, H), jnp.float32)], axis=0))
    sel1 = jnp.kron(eye8, jnp.concatenate(
        [jnp.zeros((H, H), jnp.float32), jnp.eye(H, dtype=jnp.float32)], axis=0))

    for l in range(L):
        aggA = _hop_kernel(h0p.reshape(NP, H), h1p.reshape(NP, H), srcp, dstp, z16)
        t0p, t1p = _mid_call(aggA.reshape(2, PR, LANES8), norm2p)
        aggB = _hop_kernel(t0p.reshape(NP, H), t1p.reshape(NP, H), srcp, dstp, z16)
        w0 = jnp.pad(W[l, 0:D, :], ((0, DP - D), (0, DP - D)))
        w1 = jnp.pad(W[l, D:2 * D, :], ((0, DP - D), (0, DP - D)))
        w2 = jnp.pad(W[l, 2 * D:3 * D, :], ((0, DP - D), (0, DP - D)))
        kws = [
            jnp.kron(eye8, w0),
            jnp.kron(eye8, w1[:H, :]), jnp.kron(eye8, w1[H:, :]),
            jnp.kron(eye8, w2[:H, :]), jnp.kron(eye8, w2[H:, :]),
        ]
        bP = jnp.tile(jnp.pad(b[l], (0, DP - D)), (8, 8)).reshape(8, 8 * DP)[:, :W32]
        hP, h0p, h1p = _layer_call(hP, aggA.reshape(2, PR, LANES8),
                                   aggB.reshape(2, PR, LANES8), normp, norm32p,
                                   kws, bP, sel0, sel1)

    gwp = jnp.pad(gate_W[:, 0], (0, DP - D))
    kgw = jnp.kron(eye8, gwp[:, None])
    bc32 = jnp.kron(eye8, jnp.ones((1, DP), jnp.float32))
    fold = jnp.tile(jnp.eye(DP, dtype=jnp.float32), (8, 1))
    (gmax,) = _gate_call(hP, gid8, kgw)
    rep, ssum = _pool_call(hP, gid8, kgw, bc32, fold, gmax)
    rep10 = rep[:G, :D] / ssum[:G, 0:1]

    rep_flat = rep10.reshape(-1)
    offsets = (jnp.cumsum(num) - num) * D
    flat_len = (num.shape[0] // G) * D

    def _take(o):
        row = lax.dynamic_slice(rep_flat, (o,), (flat_len,))
        return jnp.pad(row, (0, MAXL - flat_len))

    return jax.vmap(_take)(offsets)
